# baseline (speedup 1.0000x reference)
import jax
import jax.numpy as jnp
from jax.experimental import pallas as pl
from jax.experimental.pallas import tpu as pltpu

Q = 20
EPS = 1e-8
BB = 16
L = 13
K = 11


def _bf16_round(v):
    return v.astype(jnp.bfloat16).astype(jnp.float32)


def _body(x_ref, wcls_ref, mu_ref, c_ref, wcol_ref, b_ref, out_ref):
    l = pl.program_id(1)
    x = x_ref[0]
    wl = wcol_ref[0]

    dmask = jax.lax.broadcasted_iota(jnp.int32, (K, 512), 1) >= Q

    scores = []
    for i in range(BB):
        chunks = []
        for c in range(0, 512, 128):
            xc = x[i, c:c + 128, :]
            s2 = jnp.sum(xc * xc, axis=-1, keepdims=True)
            r = 1.0 / jnp.maximum(jnp.sqrt(s2), EPS)
            chunks.append((xc * r).astype(jnp.bfloat16))
        n = jnp.concatenate(chunks, axis=0)
        qn = chunks[0][:Q, :]

        sim = jax.lax.dot_general(
            qn, n, (((1,), (1,)), ((), ())),
            preferred_element_type=jnp.float32)

        g_list = []
        for k in range(K):
            e = sim - mu_ref[k]
            ek = jnp.exp(c_ref[k] * e * e)
            g_list.append(jnp.sum(ek, axis=0, keepdims=True))
        gs = jnp.concatenate(g_list, axis=0)
        gs = jnp.where(dmask, gs, 0.0)
        pooled = jnp.sum(gs, axis=-1, keepdims=True)
        pooled = _bf16_round(pooled)
        scores.append(jnp.sum(pooled * wl))

    contrib = jnp.stack(scores)[None, :]

    @pl.when(l == 0)
    def _():
        out_ref[...] = (contrib + b_ref[0])[None]

    @pl.when(l != 0)
    def _():
        out_ref[...] = out_ref[...] + contrib[None]

    @pl.when(l == L - 1)
    def _():
        cls = _bf16_round(x[:, 0, :])
        cc = jnp.sum(cls * wcls_ref[...], axis=-1)
        out_ref[...] = out_ref[...] + cc[None, None, :]


def kernel(hidden_states, mu, sigma, W_combine, b_combine):
    B = hidden_states.shape[1]
    w = W_combine[0]
    wcls = w[:768].reshape(1, 768).astype(jnp.bfloat16).astype(jnp.float32)
    wk = w[768:].reshape(L + 1, K).astype(jnp.bfloat16).astype(jnp.float32)
    w_eff = jnp.concatenate([(wk[0] + wk[1])[None, :], wk[2:]], axis=0)
    wcol = w_eff.reshape(L, K, 1)
    c = -0.5 / (sigma * sigma)

    out = pl.pallas_call(
        _body,
        out_shape=jax.ShapeDtypeStruct((B // BB, 1, BB), jnp.float32),
        grid=(B // BB, L),
        in_specs=[
            pl.BlockSpec((1, BB, 512, 768), lambda bb, l: (l, bb, 0, 0)),
            pl.BlockSpec((1, 768), lambda bb, l: (0, 0)),
            pl.BlockSpec(memory_space=pltpu.SMEM),
            pl.BlockSpec(memory_space=pltpu.SMEM),
            pl.BlockSpec((1, K, 1), lambda bb, l: (l, 0, 0)),
            pl.BlockSpec(memory_space=pltpu.SMEM),
        ],
        out_specs=pl.BlockSpec((1, 1, BB), lambda bb, l: (bb, 0, 0)),
        compiler_params=pltpu.CompilerParams(
            dimension_semantics=("parallel", "arbitrary"),
            vmem_limit_bytes=56 * 1024 * 1024,
        ),
        name="cedr_knrm",
    )(hidden_states, wcls, mu, c, wcol, b_combine)
    return out.reshape(B, 1)


# speedup vs baseline: 1.0483x; 1.0483x over previous
import jax
import jax.numpy as jnp
from jax.experimental import pallas as pl
from jax.experimental.pallas import tpu as pltpu

Q = 20
EPS = 1e-8
BB = 16
L = 13
K = 11


def _bf16_round(v):
    return v.astype(jnp.bfloat16).astype(jnp.float32)


def _body(x_ref, wcls_ref, c_ref, wcol_ref, ccol_ref, b_ref, out_ref):
    l = pl.program_id(1)
    x = x_ref[0]
    wl = wcol_ref[0]
    ccol = ccol_ref[...]

    dmask = jax.lax.broadcasted_iota(jnp.int32, (Q, 512), 1) >= Q
    c0 = c_ref[0]
    g0 = c0 * (-0.4)

    scores = []
    for i in range(BB):
        chunks = []
        for c in range(0, 512, 128):
            xc = x[i, c:c + 128, :]
            s2 = jnp.sum(xc * xc, axis=-1, keepdims=True)
            r = 1.0 / jnp.maximum(jnp.sqrt(s2), EPS)
            chunks.append((xc * r).astype(jnp.bfloat16))
        n = jnp.concatenate(chunks, axis=0)
        qn = chunks[0][:Q, :]

        sim = jax.lax.dot_general(
            qn, n, (((1,), (1,)), ((), ())),
            preferred_element_type=jnp.float32)

        sim = jnp.where(dmask, sim, 4.0)
        z = jnp.exp(c0 * sim * sim)
        t = jnp.exp(g0 * sim)
        tin = 1.0 / t
        powers = [None] * K
        powers[5] = z
        p = z
        for j in range(6, K):
            p = p * t
            powers[j] = p
        p = z
        for j in range(4, -1, -1):
            p = p * tin
            powers[j] = p
        g_list = [jnp.sum(pk, axis=0, keepdims=True) for pk in powers]
        gs = jnp.concatenate(g_list, axis=0)
        pooled = jnp.sum(gs, axis=-1, keepdims=True)
        pooled = _bf16_round(pooled * ccol)
        scores.append(jnp.sum(pooled * wl))

    contrib = jnp.stack(scores)[None, :]

    @pl.when(l == 0)
    def _():
        out_ref[...] = (contrib + b_ref[0])[None]

    @pl.when(l != 0)
    def _():
        out_ref[...] = out_ref[...] + contrib[None]

    @pl.when(l == L - 1)
    def _():
        cls = _bf16_round(x[:, 0, :])
        cc = jnp.sum(cls * wcls_ref[...], axis=-1)
        out_ref[...] = out_ref[...] + cc[None, None, :]


def kernel(hidden_states, mu, sigma, W_combine, b_combine):
    B = hidden_states.shape[1]
    w = W_combine[0]
    wcls = w[:768].reshape(1, 768).astype(jnp.bfloat16).astype(jnp.float32)
    wk = w[768:].reshape(L + 1, K).astype(jnp.bfloat16).astype(jnp.float32)
    w_eff = jnp.concatenate([(wk[0] + wk[1])[None, :], wk[2:]], axis=0)
    wcol = w_eff.reshape(L, K, 1)
    c = -0.5 / (sigma * sigma)
    ccol = jnp.exp(c * mu * mu).reshape(K, 1)

    out = pl.pallas_call(
        _body,
        out_shape=jax.ShapeDtypeStruct((B // BB, 1, BB), jnp.float32),
        grid=(B // BB, L),
        in_specs=[
            pl.BlockSpec((1, BB, 512, 768), lambda bb, l: (l, bb, 0, 0)),
            pl.BlockSpec((1, 768), lambda bb, l: (0, 0)),
            pl.BlockSpec(memory_space=pltpu.SMEM),
            pl.BlockSpec((1, K, 1), lambda bb, l: (l, 0, 0)),
            pl.BlockSpec((K, 1), lambda bb, l: (0, 0)),
            pl.BlockSpec(memory_space=pltpu.SMEM),
        ],
        out_specs=pl.BlockSpec((1, 1, BB), lambda bb, l: (bb, 0, 0)),
        compiler_params=pltpu.CompilerParams(
            dimension_semantics=("parallel", "arbitrary"),
            vmem_limit_bytes=56 * 1024 * 1024,
        ),
        name="cedr_knrm",
    )(hidden_states, wcls, c, wcol, ccol, b_combine)
    return out.reshape(B, 1)


# speedup vs baseline: 1.0822x; 1.0323x over previous
import jax
import jax.numpy as jnp
from jax.experimental import pallas as pl
from jax.experimental.pallas import tpu as pltpu

Q = 20
EPS = 1e-8
BB = 16
L = 13
K = 11


def _bf16_round(v):
    return v.astype(jnp.bfloat16).astype(jnp.float32)


def _body(x_ref, wcls_ref, c_ref, wcol_ref, ccol_ref, b_ref, out_ref):
    l = pl.program_id(1)
    x = x_ref[0]
    wl = wcol_ref[l]
    ccol = ccol_ref[...]

    dmask = jax.lax.broadcasted_iota(jnp.int32, (Q, 512), 1) >= Q
    c0 = c_ref[0]
    g0 = c0 * (-0.4)

    scores = []
    for i in range(BB):
        chunks = []
        for c in range(0, 512, 128):
            xc = x[i, c:c + 128, :]
            s2 = jnp.sum(xc * xc, axis=-1, keepdims=True)
            r = 1.0 / jnp.maximum(jnp.sqrt(s2), EPS)
            chunks.append((xc * r).astype(jnp.bfloat16))
        n = jnp.concatenate(chunks, axis=0)
        qn = chunks[0][:Q, :]

        sim = jax.lax.dot_general(
            qn, n, (((1,), (1,)), ((), ())),
            preferred_element_type=jnp.float32)

        sim = jnp.where(dmask, sim, 4.0)
        z = jnp.exp(c0 * sim * sim)
        t = jnp.exp(g0 * sim)
        tin = 1.0 / t
        powers = [None] * K
        powers[5] = z
        p = z
        for j in range(6, K):
            p = p * t
            powers[j] = p
        p = z
        for j in range(4, -1, -1):
            p = p * tin
            powers[j] = p
        g_list = [jnp.sum(pk, axis=0, keepdims=True) for pk in powers]
        gs = jnp.concatenate(g_list, axis=0)
        pooled = jnp.sum(gs, axis=-1, keepdims=True)
        pooled = _bf16_round(pooled * ccol)
        scores.append(jnp.sum(pooled * wl))

    contrib = jnp.stack(scores)[None, :]

    @pl.when(l == 0)
    def _():
        out_ref[...] = (contrib + b_ref[0])[None]

    @pl.when(l != 0)
    def _():
        out_ref[...] = out_ref[...] + contrib[None]

    @pl.when(l == L - 1)
    def _():
        cls = _bf16_round(x[:, 0, :])
        cc = jnp.sum(cls * wcls_ref[...], axis=-1)
        out_ref[...] = out_ref[...] + cc[None, None, :]


def kernel(hidden_states, mu, sigma, W_combine, b_combine):
    B = hidden_states.shape[1]
    w = W_combine[0]
    wcls = w[:768].reshape(1, 768).astype(jnp.bfloat16).astype(jnp.float32)
    wk = w[768:].reshape(L + 1, K).astype(jnp.bfloat16).astype(jnp.float32)
    w_eff = jnp.concatenate([(wk[0] + wk[1])[None, :], wk[2:]], axis=0)
    wcol = w_eff.reshape(L, K, 1)
    c = -0.5 / (sigma * sigma)
    ccol = jnp.exp(c * mu * mu).reshape(K, 1)

    out = pl.pallas_call(
        _body,
        out_shape=jax.ShapeDtypeStruct((B // BB, 1, BB), jnp.float32),
        grid=(B // BB, L),
        in_specs=[
            pl.BlockSpec((1, BB, 512, 768), lambda bb, l: (l, bb, 0, 0)),
            pl.BlockSpec((1, 768), lambda bb, l: (0, 0)),
            pl.BlockSpec(memory_space=pltpu.SMEM),
            pl.BlockSpec((L, K, 1), lambda bb, l: (0, 0, 0)),
            pl.BlockSpec((K, 1), lambda bb, l: (0, 0)),
            pl.BlockSpec(memory_space=pltpu.SMEM),
        ],
        out_specs=pl.BlockSpec((1, 1, BB), lambda bb, l: (bb, 0, 0)),
        compiler_params=pltpu.CompilerParams(
            dimension_semantics=("parallel", "arbitrary"),
            vmem_limit_bytes=56 * 1024 * 1024,
        ),
        name="cedr_knrm",
    )(hidden_states, wcls, c, wcol, ccol, b_combine)
    return out.reshape(B, 1)
